# Initial kernel scaffold; baseline (speedup 1.0000x reference)
import jax
import jax.numpy as jnp
from jax.experimental import pallas as pl


def kernel(x, enc_w1, enc_b1, enc_w2, enc_b2, enc_w3, enc_b3, node_w1, node_b1, node_w2, node_b2, ang_logit, rad_logit, ro_w1, ro_b1, ro_w2, ro_b2):
    raise NotImplementedError("write your pallas kernel here")


# speedup vs baseline: 13.4960x; 13.4960x over previous
import numpy as np
import jax
import jax.numpy as jnp
from jax.experimental import pallas as pl
from jax.experimental.pallas import tpu as pltpu

_MSG_ANG = 4
_MSG_RAD = 2
_NUM_NODES = _MSG_ANG * _MSG_RAD
_EMBED = 12
_PI = np.float32(np.pi)
_BLK = 2048


def _angular_adjacency():
    adj = np.zeros((_MSG_ANG, _MSG_ANG), np.float32)
    for i in range(_MSG_ANG):
        adj[i, (i - 1) % _MSG_ANG] = 1.0
        adj[i, (i + 1) % _MSG_ANG] = 1.0
    return adj


_ANG_ADJ = _angular_adjacency()
_RAD_ADJ = np.array([[0.0, 1.0], [1.0, 0.0]], np.float32)

_ES_T = np.zeros((_NUM_NODES * _EMBED, _NUM_NODES), np.float32)
_EZ_T = np.zeros((_NUM_NODES * _EMBED, 3), np.float32)
_ET_T = np.zeros((_NUM_NODES * _EMBED, _NUM_NODES), np.float32)
for _n in range(_NUM_NODES):
    for _d in range(3):
        _ES_T[_n * _EMBED + _d, _n] = 1.0
        _EZ_T[_n * _EMBED + _d, _d] = 1.0
    _ET_T[_n * _EMBED + 3, _n] = 1.0

_CORNERS = ((0, 0), (0, 1), (1, 0), (1, 1))


def _body(zi_ref, w1_ref, b1_ref, w2_ref, b2_ref, w3_ref, b3_ref,
          es_ref, ez_ref, et_ref, w1eff_ref, bg_ref, w2ro_ref, bv_ref,
          row2_ref, bo_ref, out_ref):
    f32 = jnp.float32
    zi = zi_ref[...]
    h1 = jnp.tanh(jnp.dot(w1_ref[...], zi, preferred_element_type=f32)
                  + b1_ref[...])
    h2 = jnp.tanh(jnp.dot(w2_ref[...], h1, preferred_element_type=f32)
                  + b2_ref[...])
    z = jnp.dot(w3_ref[...], h2, preferred_element_type=f32) + b3_ref[...]

    r_idx = jax.nn.sigmoid(z[0:1, :]) * np.float32(_MSG_RAD - 1)
    phi_idx = (z[1:2, :] + _PI) * np.float32(_MSG_ANG / (2.0 * np.pi))
    r0 = jnp.trunc(r_idx)
    p0 = jnp.trunc(phi_idx)
    dr = r_idx - r0
    dp = phi_idx - p0
    r0i = r0.astype(jnp.int32)
    p0i = p0.astype(jnp.int32)

    n_lanes = zi.shape[1]
    node_iota = jax.lax.broadcasted_iota(jnp.int32, (_NUM_NODES, n_lanes), 0)
    s_acc = jnp.zeros((_NUM_NODES, n_lanes), f32)
    t_acc = jnp.zeros((_NUM_NODES, n_lanes), f32)
    for dro, dpo in _CORNERS:
        wr = dr if dro else (1.0 - dr)
        wp = dp if dpo else (1.0 - dp)
        w = wr * wp
        pos = w > 0.0
        r_i = jnp.minimum(r0i + dro, _MSG_RAD - 1)
        p_i = jnp.mod(p0i + dpo, _MSG_ANG)
        cell = r_i * _MSG_ANG + p_i
        hit = (node_iota == cell) & pos
        s_acc = s_acc + jnp.where(hit, 1.0, 0.0)
        t_acc = t_acc + jnp.where(hit, jnp.broadcast_to(w, hit.shape), 0.0)

    h96 = (jnp.dot(es_ref[...], s_acc, preferred_element_type=f32)
           * jnp.dot(ez_ref[...], z, preferred_element_type=f32)
           + jnp.dot(et_ref[...], t_acc, preferred_element_type=f32))

    g = jnp.tanh(jnp.dot(w1eff_ref[...], h96, preferred_element_type=f32)
                 + bg_ref[...])
    v = jnp.tanh(jnp.dot(w2ro_ref[...], g, preferred_element_type=f32)
                 + bv_ref[...])
    out_ref[...] = (jnp.dot(row2_ref[...], v, preferred_element_type=f32)
                    + bo_ref[...])


def kernel(x, enc_w1, enc_b1, enc_w2, enc_b2, enc_w3, enc_b3,
           node_w1, node_b1, node_w2, node_b2, ang_logit, rad_logit,
           ro_w1, ro_b1, ro_w2, ro_b2):
    f32 = jnp.float32
    bn = x.shape[0]
    zi_t = x[:, -1, :].T

    a = jax.nn.softmax(ang_logit)[None, :] * jnp.asarray(_ANG_ADJ)
    a = a / jnp.maximum(a.sum(1, keepdims=True), 1e-6)
    r = jax.nn.softmax(rad_logit)[None, :] * jnp.asarray(_RAD_ADJ)
    r = r / jnp.maximum(r.sum(1, keepdims=True), 1e-6)
    mix = (jnp.eye(96, dtype=f32)
           + jnp.kron(a.T, jnp.eye(_NUM_NODES * _EMBED // _MSG_ANG, dtype=f32))
           + jnp.kron(r.T, jnp.eye(_NUM_NODES * _EMBED // _MSG_RAD, dtype=f32)))
    w1blk = jnp.kron(jnp.eye(_NUM_NODES, dtype=f32), node_w1)
    w1eff_t = (mix @ w1blk).T
    bg = jnp.tile(node_b1, _NUM_NODES).reshape(-1, 1)
    w2blk = jnp.kron(jnp.eye(_NUM_NODES, dtype=f32), node_w2)
    w2ro_t = (w2blk @ ro_w1).T
    bv = (jnp.tile(node_b2, _NUM_NODES) @ ro_w1 + ro_b1).reshape(-1, 1)
    row2_t = ro_w2.T
    bo = ro_b2.reshape(-1, 1)

    w1t = enc_w1.T
    b1 = enc_b1.reshape(-1, 1)
    w2t = enc_w2.T
    b2 = enc_b2.reshape(-1, 1)
    w3t = enc_w3.T
    b3 = enc_b3.reshape(-1, 1)

    blk = _BLK if bn % _BLK == 0 else bn
    grid = (bn // blk,)

    def _full(arr):
        return pl.BlockSpec(arr.shape, lambda i: (0,) * arr.ndim)

    weights = (w1t, b1, w2t, b2, w3t, b3,
               jnp.asarray(_ES_T), jnp.asarray(_EZ_T), jnp.asarray(_ET_T),
               w1eff_t, bg, w2ro_t, bv, row2_t, bo)
    out_t = pl.pallas_call(
        _body,
        grid=grid,
        in_specs=[pl.BlockSpec((4, blk), lambda i: (0, i))]
                 + [_full(w) for w in weights],
        out_specs=pl.BlockSpec((4, blk), lambda i: (0, i)),
        out_shape=jax.ShapeDtypeStruct((4, bn), f32),
        compiler_params=pltpu.CompilerParams(
            dimension_semantics=("parallel",),
        ),
    )(zi_t, *weights)
    return out_t.T
